# baseline (speedup 1.0000x reference)
import functools

import jax
import jax.numpy as jnp
from jax import lax
from jax.experimental import pallas as pl
from jax.experimental.pallas import tpu as pltpu
from jax.experimental.pallas import tpu_sc as plsc

L = 16
NC = 2
NS = 16
NW = NC * NS
G = 16


def _sc_swap_normalize(flat, y_rows, R, H):
    n_full = H // L if H % L else H // L - 1
    tail_off = H - L
    new_from = n_full * L - tail_off

    rows_per_w = R // NW
    n_groups = rows_per_w // G

    mesh = plsc.VectorSubcoreMesh(core_axis_name="c", subcore_axis_name="s")

    @functools.partial(
        pl.kernel,
        out_type=jax.ShapeDtypeStruct((R, H), jnp.float32),
        mesh=mesh,
        scratch_types=[
            pltpu.VMEM((G, H), jnp.float32),
            pltpu.VMEM((L,), jnp.int32),
        ],
        compiler_params=pltpu.CompilerParams(needs_layout_passes=False),
    )
    def k(flat_hbm, yrow_hbm, out_hbm, rows_v, y_v):
        wid = lax.axis_index("s") * NC + lax.axis_index("c")
        base = wid * rows_per_w
        iota = lax.iota(jnp.int32, L)

        def group_body(g, _):
            row0 = base + g * G
            pltpu.sync_copy(flat_hbm.at[pl.ds(row0, G)], rows_v)
            pltpu.sync_copy(yrow_hbm.at[pl.ds(row0, G)], y_v)

            for j in range(G):
                jv = jnp.full((L,), j, jnp.int32)

                def body1(c, carry):
                    s, m, idx = carry
                    off = c * L
                    v = rows_v[j, pl.ds(off, L)]
                    gi = off + iota
                    upd = v > m
                    m = jnp.where(upd, v, m)
                    idx = jnp.where(upd, gi, idx)
                    return s + jnp.abs(v), m, idx

                s0 = jnp.zeros((L,), jnp.float32)
                m0 = jnp.full((L,), -jnp.inf, jnp.float32)
                i0 = jnp.zeros((L,), jnp.int32)
                s, m, idx = lax.fori_loop(0, n_full, body1, (s0, m0, i0))
                v = rows_v[j, pl.ds(tail_off, L)]
                gi = tail_off + iota
                s = s + jnp.where(iota >= new_from, jnp.abs(v), 0.0)
                upd = v > m
                m = jnp.where(upd, v, m)
                idx = jnp.where(upd, gi, idx)

                dnums = lax.GatherDimensionNumbers(
                    offset_dims=(), collapsed_slice_dims=(0,),
                    start_index_map=(0,))

                def shuffle(v, perm):
                    return lax.gather(
                        v, perm[:, None], dnums, slice_sizes=(1,),
                        mode=lax.GatherScatterMode.PROMISE_IN_BOUNDS)

                def allred(v, op):
                    for sh in (8, 4, 2, 1):
                        v = op(v, shuffle(v, iota ^ sh))
                    return v

                l1_v = allred(s, jnp.add)
                vmax_v = allred(m, jnp.maximum)
                cand = jnp.where(m == vmax_v, idx, jnp.int32(2**30))
                amax_v = allred(cand, jnp.minimum)

                scale_v = 1.0 / jnp.maximum(l1_v, 1e-12)
                hmaxn = vmax_v * scale_v
                yj = plsc.load_gather(y_v, [jv])
                raw_label = plsc.load_gather(rows_v, [jv, yj])
                hlabn = raw_label * scale_v

                def body2(c, carry):
                    off = c * L
                    v = rows_v[j, pl.ds(off, L)]
                    gi = off + iota
                    o = v * scale_v
                    o = jnp.where(gi == yj, hmaxn, o)
                    o = jnp.where(gi == amax_v, hlabn, o)
                    rows_v[j, pl.ds(off, L)] = o
                    return carry

                lax.fori_loop(0, n_full, body2, 0)
                v = rows_v[j, pl.ds(tail_off, L)]
                gi = tail_off + iota
                o = jnp.where(iota >= new_from, v * scale_v, v)
                o = jnp.where(gi == yj, hmaxn, o)
                o = jnp.where(gi == amax_v, hlabn, o)
                rows_v[j, pl.ds(tail_off, L)] = o

            pltpu.sync_copy(rows_v, out_hbm.at[pl.ds(row0, G)])
            return _

        lax.fori_loop(0, n_groups, group_body, 0)

    return k(flat, y_rows)


def kernel(x, y, exp_sample, h_dim, sample_size):
    B, S, H = exp_sample.shape
    R = B * S
    zero = (jnp.asarray(sample_size, jnp.int32) - S) + (
        jnp.asarray(h_dim, jnp.int32) - H)
    y_idx = y.astype(jnp.int32) + zero
    y_rows = jnp.repeat(y_idx, S, total_repeat_length=R)
    flat = exp_sample.reshape(R, H)
    out = _sc_swap_normalize(flat, y_rows, R, H)
    return out.reshape(B, S, H)
